# Initial kernel scaffold; baseline (speedup 1.0000x reference)
import jax
import jax.numpy as jnp
from jax.experimental import pallas as pl


def kernel(lattice_fine_values, coarse_neighbor_indices, weight):
    raise NotImplementedError("write your pallas kernel here")


# speedup vs baseline: 1.7903x; 1.7903x over previous
import functools

import jax
import jax.numpy as jnp
from jax import lax
from jax.experimental import pallas as pl
from jax.experimental.pallas import tpu as pltpu
from jax.experimental.pallas import tpu_sc as plsc

N_FINE = 100000
N_COARSE = 25000
VAL_DIM = 128
FE = 9
NF = 128

NC_SC = 2
NS_SC = 16
NW = NC_SC * NS_SC

M_BLK = 512
M_PAD = 25088
TOT_ROWS = FE * M_PAD
CHUNK = 128
N_CHUNKS = TOT_ROWS // CHUNK
FULL_ITERS = N_CHUNKS // NW
REM = N_CHUNKS - FULL_ITERS * NW


def _sc_gather(fine, idx_flat):
    mesh = plsc.VectorSubcoreMesh(core_axis_name="c", subcore_axis_name="s")

    @functools.partial(
        pl.kernel,
        mesh=mesh,
        out_type=jax.ShapeDtypeStruct((TOT_ROWS, VAL_DIM), jnp.float32),
        scratch_types=[
            pltpu.VMEM((CHUNK,), jnp.int32),
            pltpu.VMEM((CHUNK, VAL_DIM), jnp.float32),
            pltpu.SemaphoreType.DMA,
        ],
    )
    def gather_kernel(fine_hbm, idx_hbm, out_hbm, idx_v, rows_v, sem):
        wid = lax.axis_index("s") * NC_SC + lax.axis_index("c")

        def do_chunk(chunk):
            base = pl.multiple_of(chunk * CHUNK, CHUNK)
            pltpu.sync_copy(idx_hbm.at[pl.ds(base, CHUNK)], idx_v)
            pltpu.async_copy(fine_hbm.at[idx_v], rows_v, sem).wait()
            pltpu.sync_copy(rows_v, out_hbm.at[pl.ds(base, CHUNK)])

        def body(j, carry):
            do_chunk(wid + j * NW)
            return carry

        lax.fori_loop(0, FULL_ITERS, body, 0)

        @pl.when(wid < REM)
        def _():
            do_chunk(FULL_ITERS * NW + wid)

    return gather_kernel(fine, idx_flat)


def _mm_body(a_ref, w_ref, o_ref):
    fe = pl.program_id(1)
    part = jnp.dot(a_ref[0], w_ref[0], preferred_element_type=jnp.float32)

    @pl.when(fe == 0)
    def _():
        o_ref[...] = part

    @pl.when(fe > 0)
    def _():
        o_ref[...] += part


def _tc_matmul(a3, w3):
    grid = (M_PAD // M_BLK, FE)
    return pl.pallas_call(
        _mm_body,
        grid=grid,
        in_specs=[
            pl.BlockSpec((1, M_BLK, VAL_DIM), lambda m, fe: (fe, m, 0)),
            pl.BlockSpec((1, VAL_DIM, NF), lambda m, fe: (fe, 0, 0)),
        ],
        out_specs=pl.BlockSpec((M_BLK, NF), lambda m, fe: (m, 0)),
        out_shape=jax.ShapeDtypeStruct((N_COARSE, NF), jnp.float32),
    )(a3, w3)


def kernel(lattice_fine_values, coarse_neighbor_indices, weight):
    idx = coarse_neighbor_indices.astype(jnp.int32)
    idx_t = jnp.pad(idx.T, ((0, 0), (0, M_PAD - N_COARSE)))
    idx_flat = idx_t.reshape(-1)
    gathered = _sc_gather(lattice_fine_values, idx_flat)
    a3 = gathered.reshape(FE, M_PAD, VAL_DIM)
    w3 = weight.reshape(FE, VAL_DIM, NF)
    return _tc_matmul(a3, w3)


# speedup vs baseline: 2.1288x; 1.1891x over previous
import functools

import jax
import jax.numpy as jnp
from jax import lax
from jax.experimental import pallas as pl
from jax.experimental.pallas import tpu as pltpu
from jax.experimental.pallas import tpu_sc as plsc

N_FINE = 100000
N_COARSE = 25000
VAL_DIM = 128
FE = 9
NF = 128
KDIM = FE * VAL_DIM

NC_SC = 2
NS_SC = 16
NW = NC_SC * NS_SC

M_BLK = 512
M_PAD = 25088
TOT_ROWS = FE * M_PAD
CHUNK = 128
N_CHUNKS = TOT_ROWS // CHUNK
FULL_ITERS = N_CHUNKS // NW
REM = N_CHUNKS - FULL_ITERS * NW


def _sc_gather(fine, idx_flat):
    mesh = plsc.VectorSubcoreMesh(core_axis_name="c", subcore_axis_name="s")

    @functools.partial(
        pl.kernel,
        mesh=mesh,
        out_type=jax.ShapeDtypeStruct((TOT_ROWS, VAL_DIM), jnp.float32),
        scratch_types=[
            pltpu.VMEM((CHUNK,), jnp.int32),
            pltpu.VMEM((CHUNK, VAL_DIM), jnp.float32),
            pltpu.SemaphoreType.DMA,
        ],
    )
    def gather_kernel(fine_hbm, idx_hbm, out_hbm, idx_v, rows_v, sem):
        wid = lax.axis_index("s") * NC_SC + lax.axis_index("c")

        def do_chunk(chunk):
            base = pl.multiple_of(chunk * CHUNK, CHUNK)
            pltpu.sync_copy(idx_hbm.at[pl.ds(base, CHUNK)], idx_v)
            pltpu.async_copy(fine_hbm.at[idx_v], rows_v, sem).wait()
            pltpu.sync_copy(rows_v, out_hbm.at[pl.ds(base, CHUNK)])

        def body(j, carry):
            do_chunk(wid + j * NW)
            return carry

        lax.fori_loop(0, FULL_ITERS, body, 0)

        @pl.when(wid < REM)
        def _():
            do_chunk(FULL_ITERS * NW + wid)

    return gather_kernel(fine, idx_flat)


def _mm_body(a_ref, w_ref, o_ref):
    o_ref[...] = jnp.dot(a_ref[...], w_ref[...],
                         preferred_element_type=jnp.float32)


def _tc_matmul(a2, w):
    grid = (M_PAD // M_BLK,)
    return pl.pallas_call(
        _mm_body,
        grid=grid,
        in_specs=[
            pl.BlockSpec((M_BLK, KDIM), lambda m: (m, 0)),
            pl.BlockSpec((KDIM, NF), lambda m: (0, 0)),
        ],
        out_specs=pl.BlockSpec((M_BLK, NF), lambda m: (m, 0)),
        out_shape=jax.ShapeDtypeStruct((N_COARSE, NF), jnp.float32),
    )(a2, w)


def kernel(lattice_fine_values, coarse_neighbor_indices, weight):
    idx = coarse_neighbor_indices.astype(jnp.int32)
    idx_pad = jnp.pad(idx, ((0, M_PAD - N_COARSE), (0, 0)))
    idx_flat = idx_pad.reshape(-1)
    gathered = _sc_gather(lattice_fine_values, idx_flat)
    a2 = gathered.reshape(M_PAD, KDIM)
    return _tc_matmul(a2, weight)


# speedup vs baseline: 2.5021x; 1.1753x over previous
import functools

import jax
import jax.numpy as jnp
from jax import lax
from jax.experimental import pallas as pl
from jax.experimental.pallas import tpu as pltpu
from jax.experimental.pallas import tpu_sc as plsc

N_FINE = 100000
N_COARSE = 25000
VAL_DIM = 128
FE = 9
NF = 128
KDIM = FE * VAL_DIM

NC_SC = 2
NS_SC = 16
NW = NC_SC * NS_SC

M_BLK = 512
M_PAD = 25088
TOT_ROWS = FE * M_PAD
CHUNK = 128
N_CHUNKS = TOT_ROWS // CHUNK
BASE_ITERS = N_CHUNKS // NW
REM = N_CHUNKS - BASE_ITERS * NW
MAX_ITERS = BASE_ITERS + 1
CHUNKS_PAD = NW * MAX_ITERS


def _sc_gather(fine, idx2d):
    mesh = plsc.VectorSubcoreMesh(core_axis_name="c", subcore_axis_name="s")

    @functools.partial(
        pl.kernel,
        mesh=mesh,
        out_type=jax.ShapeDtypeStruct((TOT_ROWS, VAL_DIM), jnp.float32),
        scratch_types=[
            pltpu.VMEM((MAX_ITERS + 8, CHUNK), jnp.int32),
            pltpu.VMEM((2, CHUNK, VAL_DIM), jnp.float32),
            pltpu.SemaphoreType.DMA((2,)),
            pltpu.SemaphoreType.DMA((2,)),
        ],
    )
    def gather_kernel(fine_hbm, idx_hbm, out_hbm, idx_v, rows_v, gsem, wsem):
        wid = lax.axis_index("s") * NC_SC + lax.axis_index("c")
        first = wid * BASE_ITERS + jnp.minimum(wid, REM)
        n = BASE_ITERS + (wid < REM).astype(jnp.int32)

        aligned = pl.multiple_of((first // 8) * 8, 8)
        off = first - aligned
        pltpu.sync_copy(idx_hbm.at[pl.ds(aligned, MAX_ITERS + 8)], idx_v)

        def start_gather(j, slot):
            pltpu.async_copy(fine_hbm.at[idx_v.at[j + off]], rows_v.at[slot],
                             gsem.at[slot])

        def wait_gather(slot):
            pltpu.make_async_copy(fine_hbm.at[idx_v.at[0]], rows_v.at[slot],
                                  gsem.at[slot]).wait()

        def start_write(j, slot):
            base = pl.multiple_of((first + j) * CHUNK, CHUNK)
            pltpu.async_copy(rows_v.at[slot], out_hbm.at[pl.ds(base, CHUNK)],
                             wsem.at[slot])

        def wait_write(slot):
            pltpu.make_async_copy(rows_v.at[slot],
                                  out_hbm.at[pl.ds(0, CHUNK)],
                                  wsem.at[slot]).wait()

        start_gather(0, 0)

        def body(j, carry):
            slot = lax.rem(j, 2)
            nslot = 1 - slot

            @pl.when(j + 1 < n)
            def _():
                @pl.when(j >= 1)
                def _():
                    wait_write(nslot)
                start_gather(j + 1, nslot)

            wait_gather(slot)
            start_write(j, slot)
            return carry

        lax.fori_loop(0, n, body, 0)
        @pl.when(n >= 2)
        def _():
            wait_write(lax.rem(n, 2))

        wait_write(lax.rem(n - 1, 2))

    return gather_kernel(fine, idx2d)


def _mm_body(a_ref, w_ref, o_ref):
    o_ref[...] = jnp.dot(a_ref[...], w_ref[...],
                         preferred_element_type=jnp.float32)


def _tc_matmul(a2, w):
    grid = (M_PAD // M_BLK,)
    return pl.pallas_call(
        _mm_body,
        grid=grid,
        in_specs=[
            pl.BlockSpec((M_BLK, KDIM), lambda m: (m, 0)),
            pl.BlockSpec((KDIM, NF), lambda m: (0, 0)),
        ],
        out_specs=pl.BlockSpec((M_BLK, NF), lambda m: (m, 0)),
        out_shape=jax.ShapeDtypeStruct((N_COARSE, NF), jnp.float32),
    )(a2, w)


def kernel(lattice_fine_values, coarse_neighbor_indices, weight):
    idx = coarse_neighbor_indices.astype(jnp.int32)
    idx_pad = jnp.pad(idx, ((0, M_PAD - N_COARSE), (0, 0)))
    idx2d = jnp.pad(idx_pad.reshape(-1),
                    (0, CHUNKS_PAD * CHUNK - TOT_ROWS)).reshape(CHUNKS_PAD, CHUNK)
    gathered = _sc_gather(lattice_fine_values, idx2d)
    a2 = gathered.reshape(M_PAD, KDIM)
    return _tc_matmul(a2, weight)


# speedup vs baseline: 3.9842x; 1.5923x over previous
import functools

import jax
import jax.numpy as jnp
from jax import lax
from jax.experimental import pallas as pl
from jax.experimental.pallas import tpu as pltpu
from jax.experimental.pallas import tpu_sc as plsc

N_FINE = 100000
N_COARSE = 25000
VAL_DIM = 128
FE = 9
NF = 128
KDIM = FE * VAL_DIM

NC_SC = 2
NS_SC = 16
NW = NC_SC * NS_SC

M_BLK = 512
M_PAD = 25088
TOT_ROWS = FE * M_PAD
CHUNK = 128
N_CHUNKS = TOT_ROWS // CHUNK
BASE_ITERS = N_CHUNKS // NW
REM = N_CHUNKS - BASE_ITERS * NW
MAX_ITERS = BASE_ITERS + 1
CHUNKS_PAD = NW * MAX_ITERS


def _sc_gather(fine, idx2d):
    mesh = plsc.VectorSubcoreMesh(core_axis_name="c", subcore_axis_name="s")

    @functools.partial(
        pl.kernel,
        mesh=mesh,
        out_type=jax.ShapeDtypeStruct((TOT_ROWS, VAL_DIM), jnp.float32),
        scratch_types=[
            pltpu.VMEM((MAX_ITERS + 8, CHUNK), jnp.int32),
            pltpu.VMEM((2, CHUNK, VAL_DIM), jnp.float32),
            pltpu.SemaphoreType.DMA((2,)),
            pltpu.SemaphoreType.DMA((2,)),
        ],
    )
    def gather_kernel(fine_hbm, idx_hbm, out_hbm, idx_v, rows_v, gsem, wsem):
        wid = lax.axis_index("s") * NC_SC + lax.axis_index("c")
        first = wid * BASE_ITERS + jnp.minimum(wid, REM)
        n = BASE_ITERS + (wid < REM).astype(jnp.int32)

        aligned = pl.multiple_of((first // 8) * 8, 8)
        off = first - aligned
        pltpu.sync_copy(idx_hbm.at[pl.ds(aligned, MAX_ITERS + 8)], idx_v)

        def start_gather(j, slot):
            pltpu.async_copy(fine_hbm.at[idx_v.at[j + off]], rows_v.at[slot],
                             gsem.at[slot])

        def wait_gather(slot):
            pltpu.make_async_copy(fine_hbm.at[idx_v.at[0]], rows_v.at[slot],
                                  gsem.at[slot]).wait()

        def start_write(j, slot):
            base = pl.multiple_of((first + j) * CHUNK, CHUNK)
            pltpu.async_copy(rows_v.at[slot], out_hbm.at[pl.ds(base, CHUNK)],
                             wsem.at[slot])

        def wait_write(slot):
            pltpu.make_async_copy(rows_v.at[slot],
                                  out_hbm.at[pl.ds(0, CHUNK)],
                                  wsem.at[slot]).wait()

        start_gather(0, 0)

        def body(j, carry):
            slot = lax.rem(j, 2)
            nslot = 1 - slot

            @pl.when(j + 1 < n)
            def _():
                @pl.when(j >= 1)
                def _():
                    wait_write(nslot)
                start_gather(j + 1, nslot)

            wait_gather(slot)
            start_write(j, slot)
            return carry

        lax.fori_loop(0, n, body, 0)
        @pl.when(n >= 2)
        def _():
            wait_write(lax.rem(n, 2))

        wait_write(lax.rem(n - 1, 2))

    return gather_kernel(fine, idx2d)


def _mm_body(a_ref, w_ref, o_ref):
    a = jnp.concatenate([a_ref[i] for i in range(FE)], axis=1)
    o_ref[...] = jnp.dot(a, w_ref[...], preferred_element_type=jnp.float32)


def _tc_matmul(a3, w):
    grid = (M_PAD // M_BLK,)
    return pl.pallas_call(
        _mm_body,
        grid=grid,
        in_specs=[
            pl.BlockSpec((FE, M_BLK, VAL_DIM), lambda m: (0, m, 0)),
            pl.BlockSpec((KDIM, NF), lambda m: (0, 0)),
        ],
        out_specs=pl.BlockSpec((M_BLK, NF), lambda m: (m, 0)),
        out_shape=jax.ShapeDtypeStruct((N_COARSE, NF), jnp.float32),
    )(a3, w)


def kernel(lattice_fine_values, coarse_neighbor_indices, weight):
    idx = coarse_neighbor_indices.astype(jnp.int32)
    idx_t = jnp.pad(idx.T, ((0, 0), (0, M_PAD - N_COARSE)))
    idx2d = jnp.pad(idx_t.reshape(-1),
                    (0, CHUNKS_PAD * CHUNK - TOT_ROWS)).reshape(CHUNKS_PAD, CHUNK)
    gathered = _sc_gather(lattice_fine_values, idx2d)
    a3 = gathered.reshape(FE, M_PAD, VAL_DIM)
    return _tc_matmul(a3, weight)
